# baseline (speedup 1.0000x reference)
import functools

import jax
import jax.numpy as jnp
from jax import lax
from jax.experimental import pallas as pl
from jax.experimental.pallas import tpu as pltpu
from jax.experimental.pallas import tpu_sc as plsc

NUM_GRID = 64
C = 1536
HALF = C // 2
N = 65536
L = 16
NC, NS = 2, 16
NW = NC * NS
PTS_PER_W = N // NW
CHUNK = 8
SETS = 3
NCHUNK = PTS_PER_W // CHUNK
NPAIR = NCHUNK // SETS
CPAIR = C // (2 * L)

_mesh = plsc.VectorSubcoreMesh(core_axis_name="c", subcore_axis_name="s")


@functools.partial(
    pl.kernel,
    out_type=jax.ShapeDtypeStruct((N, C), jnp.float32),
    mesh=_mesh,
    scratch_types=[
        pltpu.VMEM((PTS_PER_W,), jnp.float32),
        pltpu.VMEM((PTS_PER_W,), jnp.float32),
        pltpu.VMEM((4 * PTS_PER_W,), jnp.int32),
        pltpu.VMEM((PTS_PER_W + L,), jnp.float32),
        pltpu.VMEM((PTS_PER_W + L,), jnp.float32),
        pltpu.VMEM((4 * CHUNK, HALF), jnp.int32),
        pltpu.VMEM((4 * CHUNK, HALF), jnp.int32),
        pltpu.VMEM((4 * CHUNK, HALF), jnp.int32),
        pltpu.VMEM((CHUNK, C), jnp.float32),
        pltpu.VMEM((CHUNK, C), jnp.float32),
        pltpu.VMEM((CHUNK, C), jnp.float32),
        pltpu.SemaphoreType.DMA,
        pltpu.SemaphoreType.DMA,
        pltpu.SemaphoreType.DMA,
        pltpu.SemaphoreType.DMA,
        pltpu.SemaphoreType.DMA,
        pltpu.SemaphoreType.DMA,
    ],
)
def _decode(x_hbm, y_hbm, tab_hbm, out_hbm,
            xv, yv, ibuf, wxv, wyv,
            ca, cb, cc, oa, ob, oc,
            gsa, gsb, gsc, osa, osb, osc):
    wid = lax.axis_index("s") * NC + lax.axis_index("c")
    base = wid * PTS_PER_W
    pltpu.sync_copy(x_hbm.at[pl.ds(base, PTS_PER_W)], xv)
    pltpu.sync_copy(y_hbm.at[pl.ds(base, PTS_PER_W)], yv)

    def idx_body(i, carry):
        s = pl.ds(i * L, L)
        x = xv[s]
        y = yv[s]
        xi = jnp.clip(x.astype(jnp.int32), 0, NUM_GRID - 2)
        yi = jnp.clip(y.astype(jnp.int32), 0, NUM_GRID - 2)
        wxv[s] = x - xi.astype(jnp.float32)
        wyv[s] = y - yi.astype(jnp.float32)
        cell = xi * NUM_GRID + yi
        lane = lax.iota(jnp.int32, L)
        half = lane >> 3
        perm_lo = lane & 7
        perm_hi = perm_lo + 8
        cell_lo = cell.at[perm_lo].get(mode="promise_in_bounds")
        cell_hi = cell.at[perm_hi].get(mode="promise_in_bounds")
        o32 = i * 2 * 4 * CHUNK
        ibuf[pl.ds(o32, L)] = cell_lo + half
        ibuf[pl.ds(o32 + L, L)] = cell_lo + NUM_GRID + half
        ibuf[pl.ds(o32 + 2 * L, L)] = cell_hi + half
        ibuf[pl.ds(o32 + 3 * L, L)] = cell_hi + NUM_GRID + half
        return carry

    lax.fori_loop(0, PTS_PER_W // L, idx_body, 0, unroll=2)

    def issue_gathers(g, buf, sem):
        pltpu.async_copy(tab_hbm.at[ibuf.at[pl.ds(g * 4 * CHUNK, 4 * CHUNK)]],
                         buf, sem)

    def drain(sem, buf):
        pltpu.make_async_copy(tab_hbm.at[pl.ds(0, 4 * CHUNK)], buf, sem).wait()

    def drain_store(sem, buf):
        pltpu.make_async_copy(out_hbm.at[pl.ds(0, CHUNK)], buf, sem).wait()

    def blend(o, comb, obuf):
        wxc = wxv[pl.ds(o, L)]
        wyc = wyv[pl.ds(o, L)]
        w00c = (1.0 - wxc) * (1.0 - wyc)
        w01c = (1.0 - wxc) * wyc
        w10c = wxc * (1.0 - wyc)
        w11c = wxc * wyc
        ws = [(w00c[j], w01c[j], w10c[j], w11c[j]) for j in range(CHUNK)]
        for jj in range(0, CHUNK, 2):

            @plsc.parallel_loop(0, HALF, step=L, unroll=4)
            def col_body(ci, jj=jj):
                cs = pl.ds(ci, L)
                hs = pl.ds(HALF + ci, L)

                def widen(v):
                    lo = lax.bitcast_convert_type(v << 16, jnp.float32)
                    hi = lax.bitcast_convert_type(v, jnp.float32)
                    return lo, hi

                for j in (jj, jj + 1):
                    w00, w01, w10, w11 = ws[j]
                    l0, h0 = widen(comb[j, cs])
                    l1, h1 = widen(comb[CHUNK + j, cs])
                    l2, h2 = widen(comb[2 * CHUNK + j, cs])
                    l3, h3 = widen(comb[3 * CHUNK + j, cs])
                    obuf[j, cs] = l0 * w00 + l1 * w01 + l2 * w10 + l3 * w11
                    obuf[j, hs] = h0 * w00 + h1 * w01 + h2 * w10 + h3 * w11

    sets = ((0, ca, oa, gsa, osa),
            (1, cb, ob, gsb, osb),
            (2, cc, oc, gsc, osc))
    NTRI = NCHUNK // SETS

    for par, comb, obuf, gsem, osem in sets:
        issue_gathers(par, comb, gsem)

    def one(g, comb, obuf, gsem, osem, drain_prev, prefetch):
        o = g * CHUNK
        drain(gsem, comb)
        if drain_prev:
            drain_store(osem, obuf)
        blend(o, comb, obuf)
        pltpu.async_copy(obuf, out_hbm.at[pl.ds(base + o, CHUNK)], osem)
        if prefetch:
            issue_gathers(g + SETS, comb, gsem)

    def body(t, carry, drain_prev, prefetch):
        for par, comb, obuf, gsem, osem in sets:
            one(t * SETS + par, comb, obuf, gsem, osem, drain_prev, prefetch)
        return carry

    body(0, 0, drain_prev=False, prefetch=True)
    lax.fori_loop(1, NTRI - 1,
                  functools.partial(body, drain_prev=True, prefetch=True), 0)
    t_last = NTRI - 1
    one(t_last * SETS + 0, ca, oa, gsa, osa, drain_prev=True, prefetch=True)
    one(t_last * SETS + 1, cb, ob, gsb, osb, drain_prev=True, prefetch=False)
    one(t_last * SETS + 2, cc, oc, gsc, osc, drain_prev=True, prefetch=False)
    one(NCHUNK - 1, ca, oa, gsa, osa, drain_prev=True, prefetch=False)
    drain_store(osa, oa)
    drain_store(osb, ob)
    drain_store(osc, oc)


def kernel(x_prime, u):
    tab = u.transpose(1, 2, 0).reshape(NUM_GRID * NUM_GRID, 2, HALF)
    tab = tab.transpose(0, 2, 1).astype(jnp.bfloat16)
    tab = lax.bitcast_convert_type(tab, jnp.int32)
    return _decode(x_prime[:, 0], x_prime[:, 1], tab)


# speedup vs baseline: 1.0232x; 1.0232x over previous
import functools

import jax
import jax.numpy as jnp
from jax import lax
from jax.experimental import pallas as pl
from jax.experimental.pallas import tpu as pltpu
from jax.experimental.pallas import tpu_sc as plsc

NUM_GRID = 64
C = 1536
HALF = C // 2
N = 65536
L = 16
NC, NS = 2, 16
NW = NC * NS
PTS_PER_W = N // NW
CHUNK = 8
SETS = 3
NCHUNK = PTS_PER_W // CHUNK
NPAIR = NCHUNK // SETS
CPAIR = C // (2 * L)

_mesh = plsc.VectorSubcoreMesh(core_axis_name="c", subcore_axis_name="s")


@functools.partial(
    pl.kernel,
    out_type=jax.ShapeDtypeStruct((N, C), jnp.float32),
    mesh=_mesh,
    scratch_types=[
        pltpu.VMEM((PTS_PER_W,), jnp.float32),
        pltpu.VMEM((PTS_PER_W,), jnp.float32),
        pltpu.VMEM((4 * PTS_PER_W,), jnp.int32),
        pltpu.VMEM((PTS_PER_W + L,), jnp.float32),
        pltpu.VMEM((PTS_PER_W + L,), jnp.float32),
        pltpu.VMEM((4 * CHUNK, HALF), jnp.int32),
        pltpu.VMEM((4 * CHUNK, HALF), jnp.int32),
        pltpu.VMEM((4 * CHUNK, HALF), jnp.int32),
        pltpu.VMEM((CHUNK, C), jnp.float32),
        pltpu.VMEM((CHUNK, C), jnp.float32),
        pltpu.VMEM((CHUNK, C), jnp.float32),
        pltpu.SemaphoreType.DMA,
        pltpu.SemaphoreType.DMA,
        pltpu.SemaphoreType.DMA,
        pltpu.SemaphoreType.DMA,
        pltpu.SemaphoreType.DMA,
        pltpu.SemaphoreType.DMA,
    ],
)
def _decode(x_hbm, y_hbm, tab_hbm, out_hbm,
            xv, yv, ibuf, wxv, wyv,
            ca, cb, cc, oa, ob, oc,
            gsa, gsb, gsc, osa, osb, osc):
    wid = lax.axis_index("s") * NC + lax.axis_index("c")
    base = wid * PTS_PER_W
    pltpu.sync_copy(x_hbm.at[pl.ds(base, PTS_PER_W)], xv)
    pltpu.sync_copy(y_hbm.at[pl.ds(base, PTS_PER_W)], yv)

    def idx_body(i, carry):
        s = pl.ds(i * L, L)
        x = xv[s]
        y = yv[s]
        xi = jnp.clip(x.astype(jnp.int32), 0, NUM_GRID - 2)
        yi = jnp.clip(y.astype(jnp.int32), 0, NUM_GRID - 2)
        wxv[s] = x - xi.astype(jnp.float32)
        wyv[s] = y - yi.astype(jnp.float32)
        cell = xi * NUM_GRID + yi
        lane = lax.iota(jnp.int32, L)
        half = lane >> 3
        perm_lo = lane & 7
        perm_hi = perm_lo + 8
        cell_lo = cell.at[perm_lo].get(mode="promise_in_bounds")
        cell_hi = cell.at[perm_hi].get(mode="promise_in_bounds")
        o32 = i * 2 * 4 * CHUNK
        ibuf[pl.ds(o32, L)] = cell_lo + half
        ibuf[pl.ds(o32 + L, L)] = cell_lo + NUM_GRID + half
        ibuf[pl.ds(o32 + 2 * L, L)] = cell_hi + half
        ibuf[pl.ds(o32 + 3 * L, L)] = cell_hi + NUM_GRID + half
        return carry

    lax.fori_loop(0, PTS_PER_W // L, idx_body, 0, unroll=2)

    def issue_gathers(g, buf, sem):
        pltpu.async_copy(tab_hbm.at[ibuf.at[pl.ds(g * 4 * CHUNK, 4 * CHUNK)]],
                         buf, sem)

    def drain(sem, buf):
        pltpu.make_async_copy(tab_hbm.at[pl.ds(0, 4 * CHUNK)], buf, sem).wait()

    def drain_store(sem, buf):
        pltpu.make_async_copy(out_hbm.at[pl.ds(0, CHUNK)], buf, sem).wait()

    def blend(o, comb, obuf):
        wxc = wxv[pl.ds(o, L)]
        wyc = wyv[pl.ds(o, L)]
        w00c = (1.0 - wxc) * (1.0 - wyc)
        w01c = (1.0 - wxc) * wyc
        w10c = wxc * (1.0 - wyc)
        w11c = wxc * wyc
        ws = [(w00c[j], w01c[j], w10c[j], w11c[j]) for j in range(CHUNK)]
        for j in range(CHUNK):
            w00, w01, w10, w11 = ws[j]

            @plsc.parallel_loop(0, HALF, step=L, unroll=4)
            def col_body(ci, j=j, w00=w00, w01=w01, w10=w10, w11=w11):
                cs = pl.ds(ci, L)
                hs = pl.ds(HALF + ci, L)

                def widen(v):
                    lo = lax.bitcast_convert_type(v << 16, jnp.float32)
                    hi = lax.bitcast_convert_type(v, jnp.float32)
                    return lo, hi

                l0, h0 = widen(comb[j, cs])
                l1, h1 = widen(comb[CHUNK + j, cs])
                l2, h2 = widen(comb[2 * CHUNK + j, cs])
                l3, h3 = widen(comb[3 * CHUNK + j, cs])
                obuf[j, cs] = l0 * w00 + l1 * w01 + l2 * w10 + l3 * w11
                obuf[j, hs] = h0 * w00 + h1 * w01 + h2 * w10 + h3 * w11

    sets = ((0, ca, oa, gsa, osa),
            (1, cb, ob, gsb, osb),
            (2, cc, oc, gsc, osc))
    NTRI = NCHUNK // SETS

    for par, comb, obuf, gsem, osem in sets:
        issue_gathers(par, comb, gsem)

    def one(g, comb, obuf, gsem, osem, drain_prev, prefetch):
        o = g * CHUNK
        drain(gsem, comb)
        if drain_prev:
            drain_store(osem, obuf)
        blend(o, comb, obuf)
        pltpu.async_copy(obuf, out_hbm.at[pl.ds(base + o, CHUNK)], osem)
        if prefetch:
            issue_gathers(g + SETS, comb, gsem)

    def body(t, carry, drain_prev, prefetch):
        for par, comb, obuf, gsem, osem in sets:
            one(t * SETS + par, comb, obuf, gsem, osem, drain_prev, prefetch)
        return carry

    body(0, 0, drain_prev=False, prefetch=True)
    lax.fori_loop(1, NTRI - 1,
                  functools.partial(body, drain_prev=True, prefetch=True), 0)
    t_last = NTRI - 1
    one(t_last * SETS + 0, ca, oa, gsa, osa, drain_prev=True, prefetch=True)
    one(t_last * SETS + 1, cb, ob, gsb, osb, drain_prev=True, prefetch=False)
    one(t_last * SETS + 2, cc, oc, gsc, osc, drain_prev=True, prefetch=False)
    one(NCHUNK - 1, ca, oa, gsa, osa, drain_prev=True, prefetch=False)
    drain_store(osa, oa)
    drain_store(osb, ob)
    drain_store(osc, oc)


def kernel(x_prime, u):
    tab = u.transpose(1, 2, 0).reshape(NUM_GRID * NUM_GRID, 2, HALF)
    tab = tab.transpose(0, 2, 1).astype(jnp.bfloat16)
    tab = lax.bitcast_convert_type(tab, jnp.int32)
    return _decode(x_prime[:, 0], x_prime[:, 1], tab)


# speedup vs baseline: 1.0423x; 1.0186x over previous
import functools

import jax
import jax.numpy as jnp
from jax import lax
from jax.experimental import pallas as pl
from jax.experimental.pallas import tpu as pltpu
from jax.experimental.pallas import tpu_sc as plsc

NUM_GRID = 64
C = 1536
HALF = C // 2
N = 65536
L = 16
NC, NS = 2, 16
NW = NC * NS
PTS_PER_W = N // NW
CHUNK = 8
SETS = 3
NCHUNK = PTS_PER_W // CHUNK
NPAIR = NCHUNK // SETS
CPAIR = C // (2 * L)

_mesh = plsc.VectorSubcoreMesh(core_axis_name="c", subcore_axis_name="s")


@functools.partial(
    pl.kernel,
    out_type=jax.ShapeDtypeStruct((N, C), jnp.float32),
    mesh=_mesh,
    scratch_types=[
        pltpu.VMEM((PTS_PER_W,), jnp.float32),
        pltpu.VMEM((PTS_PER_W,), jnp.float32),
        pltpu.VMEM((4 * PTS_PER_W,), jnp.int32),
        pltpu.VMEM((PTS_PER_W + L,), jnp.float32),
        pltpu.VMEM((PTS_PER_W + L,), jnp.float32),
        pltpu.VMEM((4 * CHUNK, HALF), jnp.int32),
        pltpu.VMEM((4 * CHUNK, HALF), jnp.int32),
        pltpu.VMEM((4 * CHUNK, HALF), jnp.int32),
        pltpu.VMEM((CHUNK, C), jnp.float32),
        pltpu.VMEM((CHUNK, C), jnp.float32),
        pltpu.VMEM((CHUNK, C), jnp.float32),
        pltpu.SemaphoreType.DMA,
        pltpu.SemaphoreType.DMA,
        pltpu.SemaphoreType.DMA,
        pltpu.SemaphoreType.DMA,
        pltpu.SemaphoreType.DMA,
        pltpu.SemaphoreType.DMA,
    ],
)
def _decode(x_hbm, y_hbm, tab_hbm, out_hbm,
            xv, yv, ibuf, wxv, wyv,
            ca, cb, cc, oa, ob, oc,
            gsa, gsb, gsc, osa, osb, osc):
    wid = lax.axis_index("s") * NC + lax.axis_index("c")
    base = wid * PTS_PER_W
    pltpu.sync_copy(x_hbm.at[pl.ds(base, PTS_PER_W)], xv)
    pltpu.sync_copy(y_hbm.at[pl.ds(base, PTS_PER_W)], yv)

    def idx_body(i, carry):
        s = pl.ds(i * L, L)
        x = xv[s]
        y = yv[s]
        xi = jnp.clip(x.astype(jnp.int32), 0, NUM_GRID - 2)
        yi = jnp.clip(y.astype(jnp.int32), 0, NUM_GRID - 2)
        wxv[s] = x - xi.astype(jnp.float32)
        wyv[s] = y - yi.astype(jnp.float32)
        cell = xi * NUM_GRID + yi
        lane = lax.iota(jnp.int32, L)
        half = lane >> 3
        perm_lo = lane & 7
        perm_hi = perm_lo + 8
        cell_lo = cell.at[perm_lo].get(mode="promise_in_bounds")
        cell_hi = cell.at[perm_hi].get(mode="promise_in_bounds")
        o32 = i * 2 * 4 * CHUNK
        ibuf[pl.ds(o32, L)] = cell_lo + half
        ibuf[pl.ds(o32 + L, L)] = cell_lo + NUM_GRID + half
        ibuf[pl.ds(o32 + 2 * L, L)] = cell_hi + half
        ibuf[pl.ds(o32 + 3 * L, L)] = cell_hi + NUM_GRID + half
        return carry

    lax.fori_loop(0, PTS_PER_W // L, idx_body, 0, unroll=2)

    def issue_gathers(g, buf, sem):
        pltpu.async_copy(tab_hbm.at[ibuf.at[pl.ds(g * 4 * CHUNK, 4 * CHUNK)]],
                         buf, sem)

    def drain(sem, buf):
        pltpu.make_async_copy(tab_hbm.at[pl.ds(0, 4 * CHUNK)], buf, sem).wait()

    def drain_store(sem, buf):
        pltpu.make_async_copy(out_hbm.at[pl.ds(0, CHUNK)], buf, sem).wait()

    def blend(o, comb, obuf):
        wxc = wxv[pl.ds(o, L)]
        wyc = wyv[pl.ds(o, L)]
        w00c = (1.0 - wxc) * (1.0 - wyc)
        w01c = (1.0 - wxc) * wyc
        w10c = wxc * (1.0 - wyc)
        w11c = wxc * wyc
        ws = [(w00c[j], w01c[j], w10c[j], w11c[j]) for j in range(CHUNK)]
        for jj in range(0, CHUNK, 2):

            @plsc.parallel_loop(0, HALF, step=L, unroll=2)
            def col_body(ci, jj=jj):
                cs = pl.ds(ci, L)
                hs = pl.ds(HALF + ci, L)

                def widen(v):
                    lo = lax.bitcast_convert_type(v << 16, jnp.float32)
                    hi = lax.bitcast_convert_type(v, jnp.float32)
                    return lo, hi

                for j in (jj, jj + 1):
                    w00, w01, w10, w11 = ws[j]
                    l0, h0 = widen(comb[j, cs])
                    l1, h1 = widen(comb[CHUNK + j, cs])
                    l2, h2 = widen(comb[2 * CHUNK + j, cs])
                    l3, h3 = widen(comb[3 * CHUNK + j, cs])
                    obuf[j, cs] = l0 * w00 + l1 * w01 + l2 * w10 + l3 * w11
                    obuf[j, hs] = h0 * w00 + h1 * w01 + h2 * w10 + h3 * w11

    sets = ((0, ca, oa, gsa, osa),
            (1, cb, ob, gsb, osb),
            (2, cc, oc, gsc, osc))
    NTRI = NCHUNK // SETS

    for par, comb, obuf, gsem, osem in sets:
        issue_gathers(par, comb, gsem)

    def one(g, comb, obuf, gsem, osem, drain_prev, prefetch):
        o = g * CHUNK
        drain(gsem, comb)
        if drain_prev:
            drain_store(osem, obuf)
        blend(o, comb, obuf)
        if prefetch:
            issue_gathers(g + SETS, comb, gsem)
        pltpu.async_copy(obuf, out_hbm.at[pl.ds(base + o, CHUNK)], osem)

    def body(t, carry, drain_prev, prefetch):
        for par, comb, obuf, gsem, osem in sets:
            one(t * SETS + par, comb, obuf, gsem, osem, drain_prev, prefetch)
        return carry

    body(0, 0, drain_prev=False, prefetch=True)
    lax.fori_loop(1, NTRI - 1,
                  functools.partial(body, drain_prev=True, prefetch=True), 0)
    t_last = NTRI - 1
    one(t_last * SETS + 0, ca, oa, gsa, osa, drain_prev=True, prefetch=True)
    one(t_last * SETS + 1, cb, ob, gsb, osb, drain_prev=True, prefetch=False)
    one(t_last * SETS + 2, cc, oc, gsc, osc, drain_prev=True, prefetch=False)
    one(NCHUNK - 1, ca, oa, gsa, osa, drain_prev=True, prefetch=False)
    drain_store(osa, oa)
    drain_store(osb, ob)
    drain_store(osc, oc)


def kernel(x_prime, u):
    tab = u.transpose(1, 2, 0).reshape(NUM_GRID * NUM_GRID, 2, HALF)
    tab = tab.transpose(0, 2, 1).astype(jnp.bfloat16)
    tab = lax.bitcast_convert_type(tab, jnp.int32)
    return _decode(x_prime[:, 0], x_prime[:, 1], tab)


# speedup vs baseline: 1.0428x; 1.0004x over previous
import functools

import jax
import jax.numpy as jnp
from jax import lax
from jax.experimental import pallas as pl
from jax.experimental.pallas import tpu as pltpu
from jax.experimental.pallas import tpu_sc as plsc

NUM_GRID = 64
C = 1536
HALF = C // 2
N = 65536
L = 16
NC, NS = 2, 16
NW = NC * NS
PTS_PER_W = N // NW
CHUNK = 8
SETS = 3
NCHUNK = PTS_PER_W // CHUNK

_mesh = plsc.VectorSubcoreMesh(core_axis_name="c", subcore_axis_name="s")


@functools.partial(
    pl.kernel,
    out_type=jax.ShapeDtypeStruct((N, C), jnp.float32),
    mesh=_mesh,
    scratch_types=[
        pltpu.VMEM((PTS_PER_W,), jnp.float32),
        pltpu.VMEM((PTS_PER_W,), jnp.float32),
        pltpu.VMEM((4 * PTS_PER_W,), jnp.int32),
        pltpu.VMEM((PTS_PER_W + L,), jnp.float32),
        pltpu.VMEM((PTS_PER_W + L,), jnp.float32),
        pltpu.VMEM((4 * CHUNK, HALF), jnp.int32),
        pltpu.VMEM((4 * CHUNK, HALF), jnp.int32),
        pltpu.VMEM((4 * CHUNK, HALF), jnp.int32),
        pltpu.VMEM((CHUNK, C), jnp.float32),
        pltpu.VMEM((CHUNK, C), jnp.float32),
        pltpu.VMEM((CHUNK, C), jnp.float32),
        pltpu.SemaphoreType.DMA,
        pltpu.SemaphoreType.DMA,
        pltpu.SemaphoreType.DMA,
        pltpu.SemaphoreType.DMA,
        pltpu.SemaphoreType.DMA,
        pltpu.SemaphoreType.DMA,
    ],
)
def _decode(x_hbm, y_hbm, tab_hbm, out_hbm,
            xv, yv, ibuf, wxv, wyv,
            ca, cb, cc, oa, ob, oc,
            gsa, gsb, gsc, osa, osb, osc):
    wid = lax.axis_index("s") * NC + lax.axis_index("c")
    base = wid * PTS_PER_W
    pltpu.sync_copy(x_hbm.at[pl.ds(base, PTS_PER_W)], xv)
    pltpu.sync_copy(y_hbm.at[pl.ds(base, PTS_PER_W)], yv)

    def idx_body(i, carry):
        s = pl.ds(i * L, L)
        x = xv[s]
        y = yv[s]
        xi = jnp.clip(x.astype(jnp.int32), 0, NUM_GRID - 2)
        yi = jnp.clip(y.astype(jnp.int32), 0, NUM_GRID - 2)
        wxv[s] = x - xi.astype(jnp.float32)
        wyv[s] = y - yi.astype(jnp.float32)
        cell = xi * NUM_GRID + yi
        lane = lax.iota(jnp.int32, L)
        half = lane >> 3
        perm_lo = lane & 7
        perm_hi = perm_lo + 8
        cell_lo = cell.at[perm_lo].get(mode="promise_in_bounds")
        cell_hi = cell.at[perm_hi].get(mode="promise_in_bounds")
        o32 = i * 2 * 4 * CHUNK
        ibuf[pl.ds(o32, L)] = cell_lo + half
        ibuf[pl.ds(o32 + L, L)] = cell_lo + NUM_GRID + half
        ibuf[pl.ds(o32 + 2 * L, L)] = cell_hi + half
        ibuf[pl.ds(o32 + 3 * L, L)] = cell_hi + NUM_GRID + half
        return carry

    lax.fori_loop(0, PTS_PER_W // L, idx_body, 0, unroll=2)

    def issue_gathers(g, buf, sem):
        pltpu.async_copy(tab_hbm.at[ibuf.at[pl.ds(g * 4 * CHUNK, 4 * CHUNK)]],
                         buf, sem)

    def drain(sem, buf):
        pltpu.make_async_copy(tab_hbm.at[pl.ds(0, 4 * CHUNK)], buf, sem).wait()

    def drain_store(sem, buf):
        pltpu.make_async_copy(out_hbm.at[pl.ds(0, CHUNK)], buf, sem).wait()

    def blend(o, comb, obuf):
        wxc = wxv[pl.ds(o, L)]
        wyc = wyv[pl.ds(o, L)]
        w00c = (1.0 - wxc) * (1.0 - wyc)
        w01c = (1.0 - wxc) * wyc
        w10c = wxc * (1.0 - wyc)
        w11c = wxc * wyc
        ws = [(w00c[j], w01c[j], w10c[j], w11c[j]) for j in range(CHUNK)]
        for jj in range(0, CHUNK, 2):

            @plsc.parallel_loop(0, HALF, step=L, unroll=2)
            def col_body(ci, jj=jj):
                cs = pl.ds(ci, L)
                hs = pl.ds(HALF + ci, L)

                def widen(v):
                    lo = lax.bitcast_convert_type(v << 16, jnp.float32)
                    hi = lax.bitcast_convert_type(v, jnp.float32)
                    return lo, hi

                for j in (jj, jj + 1):
                    w00, w01, w10, w11 = ws[j]
                    l0, h0 = widen(comb[j, cs])
                    l1, h1 = widen(comb[CHUNK + j, cs])
                    l2, h2 = widen(comb[2 * CHUNK + j, cs])
                    l3, h3 = widen(comb[3 * CHUNK + j, cs])
                    obuf[j, cs] = l0 * w00 + l1 * w01 + l2 * w10 + l3 * w11
                    obuf[j, hs] = h0 * w00 + h1 * w01 + h2 * w10 + h3 * w11

    sets = ((0, ca, oa, gsa, osa),
            (1, cb, ob, gsb, osb),
            (2, cc, oc, gsc, osc))
    NTRI = NCHUNK // SETS

    for par, comb, obuf, gsem, osem in sets:
        issue_gathers(par, comb, gsem)

    def one(g, comb, obuf, gsem, osem, drain_prev, prefetch):
        o = g * CHUNK
        drain(gsem, comb)
        if drain_prev:
            drain_store(osem, obuf)
        blend(o, comb, obuf)
        if prefetch:
            issue_gathers(g + SETS, comb, gsem)
        pltpu.async_copy(obuf, out_hbm.at[pl.ds(base + o, CHUNK)], osem)

    def body(t, carry, drain_prev, prefetch):
        for par, comb, obuf, gsem, osem in sets:
            one(t * SETS + par, comb, obuf, gsem, osem, drain_prev, prefetch)
        return carry

    body(0, 0, drain_prev=False, prefetch=True)
    lax.fori_loop(1, NTRI - 1,
                  functools.partial(body, drain_prev=True, prefetch=True), 0)
    t_last = NTRI - 1
    one(t_last * SETS + 0, ca, oa, gsa, osa, drain_prev=True, prefetch=True)
    one(t_last * SETS + 1, cb, ob, gsb, osb, drain_prev=True, prefetch=False)
    one(t_last * SETS + 2, cc, oc, gsc, osc, drain_prev=True, prefetch=False)
    one(NCHUNK - 1, ca, oa, gsa, osa, drain_prev=True, prefetch=False)
    drain_store(osa, oa)
    drain_store(osb, ob)
    drain_store(osc, oc)


def kernel(x_prime, u):
    tab = u.transpose(1, 2, 0).reshape(NUM_GRID * NUM_GRID, 2, HALF)
    tab = tab.transpose(0, 2, 1).astype(jnp.bfloat16)
    tab = lax.bitcast_convert_type(tab, jnp.int32)
    return _decode(x_prime[:, 0], x_prime[:, 1], tab)
